# baseline (speedup 1.0000x reference)
import jax
import jax.numpy as jnp
import numpy as np
from jax.experimental import pallas as pl

_NQ = 8
_K = 1024
_D = 128
_TT = 2048


def _prep_kernel(cb_ref, planes_ref, cbm2_ref, c2_ref):
    for i in range(_NQ):
        cb = cb_ref[i]
        c2_ref[i, :] = jnp.sum(cb * cb, axis=1)
        cbm2_ref[i] = np.float32(-2.0) * cb
        bits = jax.lax.bitcast_convert_type(cb, jnp.uint32)
        planes_ref[i] = jnp.concatenate(
            [((bits >> np.uint32(8 * k)) & np.uint32(0xFF)).astype(jnp.bfloat16)
             for k in range(4)], axis=-1)


def _rvq_kernel(x_ref, cbm2_ref, c2_ref, planes_ref, q_ref, codes_ref, sq_ref):
    xt = x_ref[0]
    r = xt.T
    codes_rows = []
    sq_rows = []
    iota_f = jax.lax.broadcasted_iota(
        jnp.int32, (_TT, _K), 1).astype(jnp.float32)
    for i in range(_NQ):
        cbm2 = cbm2_ref[i]
        c2 = c2_ref[i, :]
        r2 = jnp.sum(r * r, axis=1, keepdims=True)
        dots2 = jnp.concatenate(
            [jax.lax.dot_general(
                r[h * 512:(h + 1) * 512], cbm2, (((1,), (1,)), ((), ())),
                preferred_element_type=jnp.float32,
                precision=jax.lax.Precision.DEFAULT)
             for h in range(_TT // 512)], axis=0)
        dist = (r2 + dots2) + c2[None, :]
        dmin = jnp.min(dist, axis=1, keepdims=True)
        cand = jnp.where(dist <= dmin, iota_f, np.float32(_K))
        idxf = jnp.min(cand, axis=1)
        codes_rows.append(idxf.astype(jnp.int32))
        onehot = (iota_f == idxf[:, None]).astype(jnp.bfloat16)
        bytes4 = jax.lax.dot_general(
            onehot, planes_ref[i], (((1,), (0,)), ((), ())),
            preferred_element_type=jnp.float32)
        bits = (bytes4[:, 0 * _D:1 * _D].astype(jnp.int32)
                | (bytes4[:, 1 * _D:2 * _D].astype(jnp.int32) << 8)
                | (bytes4[:, 2 * _D:3 * _D].astype(jnp.int32) << 16)
                | (bytes4[:, 3 * _D:4 * _D].astype(jnp.int32) << 24))
        q = jax.lax.bitcast_convert_type(bits, jnp.float32)
        r = r - q
        sq_rows.append(jnp.sum(r * r, axis=0))
    q_ref[0] = xt - r.T
    codes_ref[:, 0, 0, :] = jnp.stack(codes_rows, axis=0)
    sq_ref[0, 0] = jnp.stack(sq_rows, axis=0)


def kernel(x, sample_rate, codebooks):
    B, D, T = x.shape
    grid = (B, T // _TT)
    planes, cbm2, c2 = pl.pallas_call(
        _prep_kernel,
        out_shape=[
            jax.ShapeDtypeStruct((_NQ, _K, 4 * _D), jnp.bfloat16),
            jax.ShapeDtypeStruct((_NQ, _K, _D), jnp.float32),
            jax.ShapeDtypeStruct((_NQ, _K), jnp.float32),
        ],
    )(codebooks)
    q, codes4, sq = pl.pallas_call(
        _rvq_kernel,
        grid=grid,
        in_specs=[
            pl.BlockSpec((1, D, _TT), lambda b, t: (b, 0, t)),
            pl.BlockSpec((_NQ, _K, _D), lambda b, t: (0, 0, 0)),
            pl.BlockSpec((_NQ, _K), lambda b, t: (0, 0)),
            pl.BlockSpec((_NQ, _K, 4 * _D), lambda b, t: (0, 0, 0)),
        ],
        out_specs=[
            pl.BlockSpec((1, D, _TT), lambda b, t: (b, 0, t)),
            pl.BlockSpec((_NQ, 1, 1, _TT), lambda b, t: (0, b, 0, t)),
            pl.BlockSpec((1, 1, _NQ, _D), lambda b, t: (b, t, 0, 0)),
        ],
        out_shape=[
            jax.ShapeDtypeStruct((B, D, T), jnp.float32),
            jax.ShapeDtypeStruct((_NQ, B, 1, T), jnp.int32),
            jax.ShapeDtypeStruct((B, T // _TT, _NQ, _D), jnp.float32),
        ],
    )(x, cbm2, c2, planes)
    codes = codes4.reshape(_NQ, B, T)
    commit_loss = jnp.sum(sq) / np.float32(_NQ * B * T * D)
    bw = jnp.asarray(
        _NQ * (np.log2(_K) * jnp.asarray(sample_rate).astype(jnp.float32) / 1000.0),
        dtype=jnp.float32)
    return q, codes, bw, commit_loss


# speedup vs baseline: 1.1077x; 1.1077x over previous
import jax
import jax.numpy as jnp
import numpy as np
from jax.experimental import pallas as pl

_NQ = 8
_K = 1024
_D = 128
_TT = 1024


def _prep_kernel(cb_ref, planes_ref, cbm2_ref, c2_ref):
    for i in range(_NQ):
        cb = cb_ref[i]
        c2_ref[i, :] = jnp.sum(cb * cb, axis=1)
        cbm2_ref[i] = np.float32(-2.0) * cb
        bits = jax.lax.bitcast_convert_type(cb, jnp.uint32)
        planes_ref[i] = jnp.concatenate(
            [((bits >> np.uint32(8 * k)) & np.uint32(0xFF)).astype(jnp.bfloat16)
             for k in range(4)], axis=-1)


def _rvq_kernel(x_ref, cbm2_ref, c2_ref, planes_ref, q_ref, codes_ref, sq_ref):
    xt = x_ref[0]
    r = xt.T
    codes_rows = []
    sq_rows = []
    iota_f = jax.lax.broadcasted_iota(
        jnp.int32, (_TT, _K), 1).astype(jnp.float32)
    for i in range(_NQ):
        cbm2 = cbm2_ref[i]
        c2 = c2_ref[i, :]
        r2 = jnp.sum(r * r, axis=1, keepdims=True)
        dots2 = jnp.concatenate(
            [jax.lax.dot_general(
                r[h * 512:(h + 1) * 512], cbm2, (((1,), (1,)), ((), ())),
                preferred_element_type=jnp.float32,
                precision=jax.lax.Precision.DEFAULT)
             for h in range(_TT // 512)], axis=0)
        dist = (r2 + dots2) + c2[None, :]
        dmin = jnp.min(dist, axis=1, keepdims=True)
        cand = jnp.where(dist <= dmin, iota_f, np.float32(_K))
        idxf = jnp.min(cand, axis=1)
        codes_rows.append(idxf.astype(jnp.int32))
        onehot = (iota_f == idxf[:, None]).astype(jnp.bfloat16)
        bytes4 = jax.lax.dot_general(
            onehot, planes_ref[i], (((1,), (0,)), ((), ())),
            preferred_element_type=jnp.float32)
        bits = (bytes4[:, 0 * _D:1 * _D].astype(jnp.int32)
                | (bytes4[:, 1 * _D:2 * _D].astype(jnp.int32) << 8)
                | (bytes4[:, 2 * _D:3 * _D].astype(jnp.int32) << 16)
                | (bytes4[:, 3 * _D:4 * _D].astype(jnp.int32) << 24))
        q = jax.lax.bitcast_convert_type(bits, jnp.float32)
        r = r - q
        sq_rows.append(jnp.sum(r * r, axis=0))
    q_ref[0] = xt - r.T
    codes_ref[:, 0, 0, :] = jnp.stack(codes_rows, axis=0)
    sq_ref[0, 0] = jnp.stack(sq_rows, axis=0)


def kernel(x, sample_rate, codebooks):
    B, D, T = x.shape
    grid = (B, T // _TT)
    planes, cbm2, c2 = pl.pallas_call(
        _prep_kernel,
        out_shape=[
            jax.ShapeDtypeStruct((_NQ, _K, 4 * _D), jnp.bfloat16),
            jax.ShapeDtypeStruct((_NQ, _K, _D), jnp.float32),
            jax.ShapeDtypeStruct((_NQ, _K), jnp.float32),
        ],
    )(codebooks)
    q, codes4, sq = pl.pallas_call(
        _rvq_kernel,
        grid=grid,
        in_specs=[
            pl.BlockSpec((1, D, _TT), lambda b, t: (b, 0, t)),
            pl.BlockSpec((_NQ, _K, _D), lambda b, t: (0, 0, 0)),
            pl.BlockSpec((_NQ, _K), lambda b, t: (0, 0)),
            pl.BlockSpec((_NQ, _K, 4 * _D), lambda b, t: (0, 0, 0)),
        ],
        out_specs=[
            pl.BlockSpec((1, D, _TT), lambda b, t: (b, 0, t)),
            pl.BlockSpec((_NQ, 1, 1, _TT), lambda b, t: (0, b, 0, t)),
            pl.BlockSpec((1, 1, _NQ, _D), lambda b, t: (b, t, 0, 0)),
        ],
        out_shape=[
            jax.ShapeDtypeStruct((B, D, T), jnp.float32),
            jax.ShapeDtypeStruct((_NQ, B, 1, T), jnp.int32),
            jax.ShapeDtypeStruct((B, T // _TT, _NQ, _D), jnp.float32),
        ],
    )(x, cbm2, c2, planes)
    codes = codes4.reshape(_NQ, B, T)
    commit_loss = jnp.sum(sq) / np.float32(_NQ * B * T * D)
    bw = jnp.asarray(
        _NQ * (np.log2(_K) * jnp.asarray(sample_rate).astype(jnp.float32) / 1000.0),
        dtype=jnp.float32)
    return q, codes, bw, commit_loss
